# Initial kernel scaffold; baseline (speedup 1.0000x reference)
import jax
import jax.numpy as jnp
from jax.experimental import pallas as pl


def kernel(x, edge_index, W, b):
    raise NotImplementedError("write your pallas kernel here")


# speedup vs baseline: 7.6453x; 7.6453x over previous
import functools

import jax
import jax.numpy as jnp
from jax import lax
from jax.experimental import pallas as pl
from jax.experimental.pallas import tpu as pltpu
from jax.experimental.pallas import tpu_sc as plsc

N_NODES = 10000
N_EDGES = 320000
D = 128
DH = D // 2

NC = 2
NS = 16
CH = 128
CPW = 158
EPW = CPW * CH
E_PAD = NS * EPW
N_PAD = 10240
STRIPE = N_PAD // NS
DUMMY = N_NODES


def _sc_aggregate(xs, src_p, dst_p, z_rows, z_deg, ones_rows):
  mesh = plsc.VectorSubcoreMesh(core_axis_name="c", subcore_axis_name="s")

  @functools.partial(
      pl.kernel,
      out_type=(
          jax.ShapeDtypeStruct((NC, N_PAD, DH), jnp.float32),
          jax.ShapeDtypeStruct((N_PAD, 16), jnp.float32),
      ),
      mesh=mesh,
      compiler_params=pltpu.CompilerParams(use_tc_tiling_on_sc=False),
      scratch_types=[
          pltpu.VMEM((CPW, CH), jnp.int32),
          pltpu.VMEM((CPW, CH), jnp.int32),
          pltpu.VMEM((CH, DH), jnp.float32),
          pltpu.VMEM((CH, DH), jnp.float32),
          pltpu.VMEM((CH, 16), jnp.float32),
          pltpu.VMEM((CH, DH), jnp.float32),
          pltpu.VMEM((STRIPE, 16), jnp.float32),
          pltpu.VMEM_SHARED((N_PAD, DH), jnp.float32),
          pltpu.VMEM_SHARED((N_PAD, 16), jnp.float32),
          pltpu.SemaphoreType.DMA,
          pltpu.SemaphoreType.DMA,
      ],
  )
  def body(x_hbm, src_hbm, dst_hbm, zrow_hbm, zdeg_hbm, ones_hbm,
           agg_out, deg_out,
           src_v, dst_v, rows0, rows1, ones_v, zrow_v, zdeg_v,
           acc, dacc, sem0, sem1):
    c = lax.axis_index("c")
    s = lax.axis_index("s")

    pltpu.sync_copy(src_hbm.at[s], src_v)
    pltpu.sync_copy(dst_hbm.at[s], dst_v)
    pltpu.sync_copy(zrow_hbm, zrow_v)
    pltpu.sync_copy(zdeg_hbm, zdeg_v)
    pltpu.sync_copy(ones_hbm, ones_v)

    base = s * STRIPE
    for t in range(STRIPE // CH):
      pltpu.sync_copy(zrow_v, acc.at[pl.ds(base + t * CH, CH)])
    pltpu.sync_copy(zdeg_v, dacc.at[pl.ds(base, STRIPE)])
    plsc.subcore_barrier()

    rows = (rows0, rows1)
    sems = (sem0, sem1)
    x_half = x_hbm.at[c]

    pltpu.async_copy(x_half.at[src_v.at[0]], rows0, sem0)

    @pl.loop(0, CPW, step=2)
    def _(g):
      for bph in range(2):
        j = g + bph

        @pl.when(j + 1 < CPW)
        def _():
          pltpu.async_copy(
              x_half.at[src_v.at[j + 1]], rows[1 - bph], sems[1 - bph])

        pltpu.make_async_copy(
            x_half.at[src_v.at[j]], rows[bph], sems[bph]).wait()
        pltpu.sync_copy(rows[bph], acc.at[dst_v.at[j]], add=True)

        @pl.when(c == 0)
        def _():
          pltpu.sync_copy(ones_v, dacc.at[dst_v.at[j]], add=True)

    plsc.subcore_barrier()

    for t in range(STRIPE // CH):
      pltpu.sync_copy(acc.at[pl.ds(base + t * CH, CH)], rows0)
      pltpu.sync_copy(rows0, agg_out.at[c, pl.ds(base + t * CH, CH)])

    @pl.when(c == 0)
    def _():
      pltpu.sync_copy(dacc.at[pl.ds(base, STRIPE)], zdeg_v)
      pltpu.sync_copy(zdeg_v, deg_out.at[pl.ds(base, STRIPE)])

  return body(xs, src_p, dst_p, z_rows, z_deg, ones_rows)


_BR = 1000


def _tc_body(a0, a1, d, w_ref, b_ref, out_ref):
  inv = 1.0 / jnp.maximum(d[0][:, 0:1], 1.0)
  h0 = a0[0] * inv
  h1 = a1[0] * inv
  w = w_ref[...]
  out_ref[...] = (
      jnp.dot(h0, w[:DH], preferred_element_type=jnp.float32)
      + jnp.dot(h1, w[DH:], preferred_element_type=jnp.float32)
      + b_ref[...])


def _tc_finish(agg, deg, W, b2):
  return pl.pallas_call(
      _tc_body,
      grid=(N_NODES // _BR,),
      in_specs=[
          pl.BlockSpec((1, _BR, DH), lambda i: (0, i, 0)),
          pl.BlockSpec((1, _BR, DH), lambda i: (1, i, 0)),
          pl.BlockSpec((1, _BR, 16), lambda i: (0, i, 0)),
          pl.BlockSpec((D, D), lambda i: (0, 0)),
          pl.BlockSpec((1, D), lambda i: (0, 0)),
      ],
      out_specs=pl.BlockSpec((_BR, D), lambda i: (i, 0)),
      out_shape=jax.ShapeDtypeStruct((N_NODES, D), jnp.float32),
  )(agg, agg, deg.reshape(1, N_PAD, 16), W, b2)


def kernel(x, edge_index, W, b):
  ei = edge_index.astype(jnp.int32)
  pad = E_PAD - N_EDGES
  src_p = jnp.concatenate(
      [ei[0], jnp.zeros((pad,), jnp.int32)]).reshape(NS, CPW, CH)
  dst_p = jnp.concatenate(
      [ei[1], jnp.full((pad,), DUMMY, jnp.int32)]).reshape(NS, CPW, CH)
  xs = x.reshape(N_NODES, NC, DH).transpose(1, 0, 2)
  z_rows = jnp.zeros((CH, DH), jnp.float32)
  z_deg = jnp.zeros((STRIPE, 16), jnp.float32)
  ones_rows = jnp.ones((CH, 16), jnp.float32)
  agg, deg = _sc_aggregate(xs, src_p, dst_p, z_rows, z_deg, ones_rows)
  return _tc_finish(agg, deg, W, b.reshape(1, D))
